# Initial kernel scaffold; baseline (speedup 1.0000x reference)
import jax
import jax.numpy as jnp
from jax.experimental import pallas as pl


def kernel(inputs, inputs_up, inputs_down, targets, epoch, features, features_up, features_down):
    raise NotImplementedError("write your pallas kernel here")


# speedup vs baseline: 3.3032x; 3.3032x over previous
import functools

import jax
import jax.numpy as jnp
from jax.experimental import pallas as pl
from jax.experimental.pallas import tpu as pltpu

TEMP = 0.05
LAMBDA2 = 0.5
INV_T = 1.0 / TEMP
BM = 1024
LANES = 128


def _tc_body(x_ref, xu_ref, xd_ref, t_ref, f_ref, fu_ref, fd_ref,
             out_ref, xn_ref, acc_ref, *, m_total, nblk):
    i = pl.program_id(0)
    b = x_ref.shape[0]

    @pl.when(i == 0)
    def _init():
        acc_ref[...] = jnp.zeros_like(acc_ref)
        for p, xr in enumerate((x_ref, xu_ref, xd_ref)):
            xv = xr[...]
            nrm = jnp.sqrt(jnp.sum(xv * xv, axis=1, keepdims=True))
            xn_ref[p, :, :] = xv / jnp.maximum(nrm, 1e-12)

    cols = i * BM + jax.lax.broadcasted_iota(jnp.int32, (b, BM), 1)
    colmask = cols < m_total
    tmask = cols == t_ref[...]

    def lanes_sum(v):
        r = v[:, 0:LANES]
        for k in range(1, BM // LANES):
            r = r + v[:, k * LANES:(k + 1) * LANES]
        return r

    for p, fr in enumerate((f_ref, fu_ref, fd_ref)):
        xn = xn_ref[p, :, :]
        s = jax.lax.dot_general(xn, fr[...], (((1,), (1,)), ((), ())),
                                preferred_element_type=jnp.float32)
        el = jnp.where(colmask, jnp.exp(INV_T * s - INV_T), 0.0)
        cd = jnp.sqrt(jnp.maximum(2.0 - 2.0 * s, 0.0))
        e1 = jnp.where(colmask, jnp.exp(cd - 2.0), 0.0)
        e2 = e1 * e1
        st = jnp.where(tmask, s, 0.0)
        acc_ref[p, 0, :, :] += lanes_sum(el)
        acc_ref[p, 1, :, :] += lanes_sum(e1)
        acc_ref[p, 2, :, :] += lanes_sum(e2)
        acc_ref[p, 3, :, :] += lanes_sum(e2 * e1)
        acc_ref[p, 4, :, :] += lanes_sum(e2 * e2)
        acc_ref[p, 5, :, :] += lanes_sum(st)

    @pl.when(i == nblk - 1)
    def _finish():
        m_f = jnp.float32(m_total)
        loss = jnp.float32(0.0)
        weights = (1.0 - LAMBDA2, LAMBDA2, LAMBDA2)
        for p in range(3):
            se_l = jnp.sum(acc_ref[p, 0, :, :], axis=1)
            a1 = jnp.sum(acc_ref[p, 1, :, :], axis=1)
            a2 = jnp.sum(acc_ref[p, 2, :, :], axis=1)
            a3 = jnp.sum(acc_ref[p, 3, :, :], axis=1)
            a4 = jnp.sum(acc_ref[p, 4, :, :], axis=1)
            st = jnp.sum(acc_ref[p, 5, :, :], axis=1)
            lse_l = INV_T + jnp.log(se_l)
            u = 1.0 / a1
            u2 = u * u
            t1 = a1 * u
            t2 = a2 * u2
            t3 = a3 * u2 * u
            t4 = a4 * u2 * u2
            delta = t1 + 0.5 * t2 + (1.0 / 6.0) * t3 + (1.0 / 24.0) * t4
            log_s = jnp.log(m_f + delta)
            cdt = jnp.sqrt(jnp.maximum(2.0 - 2.0 * st, 0.0))
            pt = jnp.exp(cdt - 2.0) * u
            ce_out = jnp.mean(lse_l - INV_T * st)
            ce_soft = jnp.mean(log_s - pt)
            loss = loss + weights[p] * (ce_out + ce_soft)
        out_ref[...] = jnp.full((1, 1), loss, jnp.float32)


def _fused_loss(x, xu, xd, t2d, f, fu, fd, *, interpret=False):
    b, d = x.shape
    m = f.shape[0]
    nblk = (m + BM - 1) // BM
    body = functools.partial(_tc_body, m_total=m, nblk=nblk)
    out = pl.pallas_call(
        body,
        grid=(nblk,),
        in_specs=[
            pl.BlockSpec((b, d), lambda i: (0, 0)),
            pl.BlockSpec((b, d), lambda i: (0, 0)),
            pl.BlockSpec((b, d), lambda i: (0, 0)),
            pl.BlockSpec((b, 1), lambda i: (0, 0)),
            pl.BlockSpec((BM, d), lambda i: (i, 0)),
            pl.BlockSpec((BM, d), lambda i: (i, 0)),
            pl.BlockSpec((BM, d), lambda i: (i, 0)),
        ],
        out_specs=pl.BlockSpec((1, 1), lambda i: (0, 0)),
        out_shape=jax.ShapeDtypeStruct((1, 1), jnp.float32),
        scratch_shapes=[
            pltpu.VMEM((3, b, d), jnp.float32),
            pltpu.VMEM((3, 6, b, LANES), jnp.float32),
        ],
        compiler_params=pltpu.CompilerParams(
            dimension_semantics=("arbitrary",),
        ),
        interpret=interpret,
    )(x, xu, xd, t2d, f, fu, fd)
    return out[0, 0]


def kernel(inputs, inputs_up, inputs_down, targets, epoch,
           features, features_up, features_down):
    del epoch
    t2d = jnp.asarray(targets, jnp.int32).reshape(-1, 1)
    return _fused_loss(inputs, inputs_up, inputs_down, t2d,
                       features, features_up, features_down)


# speedup vs baseline: 4.5422x; 1.3751x over previous
import functools

import jax
import jax.numpy as jnp
from jax.experimental import pallas as pl
from jax.experimental.pallas import tpu as pltpu
from jax.experimental.pallas import tpu_sc as plsc

TEMP = 0.05
LAMBDA2 = 0.5
INV_T = 1.0 / TEMP
BM = 1024
LANES = 128
GATHER_WINDOW = 128


def _gather_rows(f, fu, fd, idx_row):
    n = idx_row.shape[1]
    d = f.shape[1]
    mesh = plsc.VectorSubcoreMesh(core_axis_name="c", subcore_axis_name="s")
    out_t = [jax.ShapeDtypeStruct((n, d), f.dtype)] * 3

    @pl.kernel(out_type=out_t, mesh=mesh, scratch_types=[])
    def gather_kernel(f_hbm, fu_hbm, fd_hbm, i_hbm, o1, o2, o3):
        for src, dst in ((f_hbm, o1), (fu_hbm, o2), (fd_hbm, o3)):
            def body(i_vmem, o_vmem, *, src_ref=src):
                pltpu.sync_copy(src_ref.at[i_vmem.at[0]], o_vmem)

            pltpu.emit_pipeline(
                body,
                grid=(n // GATHER_WINDOW,),
                in_specs=[pl.BlockSpec((1, GATHER_WINDOW), lambda i: (0, i))],
                out_specs=[pl.BlockSpec((GATHER_WINDOW, d), lambda i: (i, 0))],
                core_axis_name=("c", "s"),
                dimension_semantics=(pltpu.PARALLEL,),
            )(i_hbm, dst)

    return gather_kernel(f, fu, fd, idx_row)


def _stream_body(x_ref, xu_ref, xd_ref, f_ref, fu_ref, fd_ref,
                 acc_ref, xn_ref, *, m_total, nblk):
    i = pl.program_id(0)
    b = x_ref.shape[0]

    @pl.when(i == 0)
    def _init():
        acc_ref[...] = jnp.zeros_like(acc_ref)
        for p, xr in enumerate((x_ref, xu_ref, xd_ref)):
            xv = xr[...]
            nrm = jnp.sqrt(jnp.sum(xv * xv, axis=1, keepdims=True))
            xn_ref[p, :, :] = xv / jnp.maximum(nrm, 1e-12)

    def lanes_sum(v):
        r = v[:, 0:LANES]
        for k in range(1, BM // LANES):
            r = r + v[:, k * LANES:(k + 1) * LANES]
        return r

    def step(masked):
        if masked:
            cols = i * BM + jax.lax.broadcasted_iota(jnp.int32, (b, BM), 1)
            colmask = cols < m_total
        for p, fr in enumerate((f_ref, fu_ref, fd_ref)):
            xn = xn_ref[p, :, :]
            s = jax.lax.dot_general(xn, fr[...], (((1,), (1,)), ((), ())),
                                    preferred_element_type=jnp.float32)
            el = jnp.exp(INV_T * s - INV_T)
            cd = jnp.sqrt(jnp.maximum(2.0 - 2.0 * s, 0.0))
            e1 = jnp.exp(cd - 2.0)
            if masked:
                el = jnp.where(colmask, el, 0.0)
                e1 = jnp.where(colmask, e1, 0.0)
            acc_ref[p, 0, :, :] += lanes_sum(el)
            acc_ref[p, 1, :, :] += lanes_sum(e1)
            acc_ref[p, 2, :, :] += lanes_sum(e1 * e1)

    @pl.when(i < nblk - 1)
    def _full():
        step(False)

    @pl.when(i == nblk - 1)
    def _tail():
        step(True)


def _finish_body(x_ref, xu_ref, xd_ref, g_ref, gu_ref, gd_ref, acc_ref,
                 out_ref, *, m_total):
    m_f = jnp.float32(m_total)
    loss = jnp.float32(0.0)
    weights = (1.0 - LAMBDA2, LAMBDA2, LAMBDA2)
    for p, (xr, gr) in enumerate(((x_ref, g_ref), (xu_ref, gu_ref),
                                  (xd_ref, gd_ref))):
        xv = xr[...]
        nrm = jnp.sqrt(jnp.sum(xv * xv, axis=1, keepdims=True))
        xn = xv / jnp.maximum(nrm, 1e-12)
        st = jnp.sum(xn * gr[...], axis=1)
        se_l = jnp.sum(acc_ref[p, 0, :, :], axis=1)
        a1 = jnp.sum(acc_ref[p, 1, :, :], axis=1)
        a2 = jnp.sum(acc_ref[p, 2, :, :], axis=1)
        lse_l = INV_T + jnp.log(se_l)
        u = 1.0 / a1
        delta = a1 * u + 0.5 * a2 * u * u
        log_s = jnp.log(m_f + delta)
        cdt = jnp.sqrt(jnp.maximum(2.0 - 2.0 * st, 0.0))
        pt = jnp.exp(cdt - 2.0) * u
        ce_out = jnp.mean(lse_l - INV_T * st)
        ce_soft = jnp.mean(log_s - pt)
        loss = loss + weights[p] * (ce_out + ce_soft)
    out_ref[...] = jnp.full((1, 1), loss, jnp.float32)


def _fused_loss(x, xu, xd, tgt, f, fu, fd, *, interpret=False):
    b, d = x.shape
    m = f.shape[0]
    nblk = (m + BM - 1) // BM

    acc = pl.pallas_call(
        functools.partial(_stream_body, m_total=m, nblk=nblk),
        grid=(nblk,),
        in_specs=[
            pl.BlockSpec((b, d), lambda i: (0, 0)),
            pl.BlockSpec((b, d), lambda i: (0, 0)),
            pl.BlockSpec((b, d), lambda i: (0, 0)),
            pl.BlockSpec((BM, d), lambda i: (i, 0)),
            pl.BlockSpec((BM, d), lambda i: (i, 0)),
            pl.BlockSpec((BM, d), lambda i: (i, 0)),
        ],
        out_specs=pl.BlockSpec((3, 3, b, LANES), lambda i: (0, 0, 0, 0)),
        out_shape=jax.ShapeDtypeStruct((3, 3, b, LANES), jnp.float32),
        scratch_shapes=[pltpu.VMEM((3, b, d), jnp.float32)],
        compiler_params=pltpu.CompilerParams(
            dimension_semantics=("arbitrary",),
        ),
        interpret=interpret,
    )(x, xu, xd, f, fu, fd)

    if interpret:
        g = jnp.take(f, tgt, axis=0)
        gu = jnp.take(fu, tgt, axis=0)
        gd = jnp.take(fd, tgt, axis=0)
    else:
        g, gu, gd = _gather_rows(f, fu, fd, tgt.reshape(1, -1))

    out = pl.pallas_call(
        functools.partial(_finish_body, m_total=m),
        grid=(1,),
        in_specs=[pl.BlockSpec((b, d), lambda i: (0, 0))] * 6 + [
            pl.BlockSpec((3, 3, b, LANES), lambda i: (0, 0, 0, 0)),
        ],
        out_specs=pl.BlockSpec((1, 1), lambda i: (0, 0)),
        out_shape=jax.ShapeDtypeStruct((1, 1), jnp.float32),
        interpret=interpret,
    )(x, xu, xd, g, gu, gd, acc)
    return out[0, 0]


def kernel(inputs, inputs_up, inputs_down, targets, epoch,
           features, features_up, features_down):
    del epoch
    tgt = jnp.asarray(targets, jnp.int32)
    return _fused_loss(inputs, inputs_up, inputs_down, tgt,
                       features, features_up, features_down)
